# baseline (speedup 1.0000x reference)
import jax
import jax.numpy as jnp
from jax.experimental import pallas as pl
from jax.experimental.pallas import tpu as pltpu

_E = 8
_DIM = 96
_HID = 384
_OUT = 96
_B = 2
_N = 1024
_NT = _B * _N

_CT = (((1,), (1,)), ((), ()))


def _row_to_col(row):
    i = jax.lax.broadcasted_iota(jnp.int32, (_E, _E), 0)
    j = jax.lax.broadcasted_iota(jnp.int32, (_E, _E), 1)
    b = jnp.broadcast_to(row, (_E, _E))
    return jnp.sum(jnp.where(i == j, b, 0.0), axis=1, keepdims=True)


def _shift_down(a):
    return jnp.concatenate([jnp.zeros((1, _HID), jnp.float32), a[:-1]], axis=0)


def _shift_up(a):
    return jnp.concatenate([a[1:], jnp.zeros((1, _HID), jnp.float32)], axis=0)


def _moe_kernel(x_ref, pp_ref, dwt_ref, fc1_ref, fc2_ref, out_ref):
    x = x_ref[...]
    wg = pp_ref[0:_E, 0:_DIM]
    wgb = _row_to_col(pp_ref[8:9, 0:_E])
    fc1b = pp_ref[9:10, :]
    dwb = pp_ref[10:11, :]
    fc2b = pp_ref[11:12, 0:_OUT]
    taps = jnp.transpose(dwt_ref[...])[3:9]
    lt = jax.lax.dot_general(wg, x, _CT, preferred_element_type=jnp.float32)
    lt = lt + wgb
    mx = jnp.max(lt, axis=0, keepdims=True)
    iota_s = jax.lax.broadcasted_iota(jnp.int32, (_E, _NT), 0)
    idx = jnp.min(jnp.where(lt == mx, iota_s, _E), axis=0, keepdims=True)
    onehot = jnp.where(iota_s == idx, 1.0, 0.0)
    pres0 = jnp.max(onehot[:, :_N], axis=1, keepdims=True)
    pres1 = jnp.max(onehot[:, _N:], axis=1, keepdims=True)
    e0 = jnp.exp(lt[:, 0:_E] - mx[:, 0:_E])
    e1 = jnp.exp(lt[:, _N:_N + _E] - mx[:, _N:_N + _E])
    p0r0 = e0[0:1] / jnp.sum(e0, axis=0, keepdims=True)
    p0r1 = e1[0:1] / jnp.sum(e1, axis=0, keepdims=True)
    masked0 = _row_to_col(p0r0) * pres0
    masked1 = _row_to_col(p0r1) * pres1
    denom = masked0 + masked1 + 1e-6
    gs_col = jnp.concatenate([masked0 / denom, masked1 / denom],
                             axis=0) * float(_B)
    x64 = jnp.concatenate([x[0:64], x[_N:_N + 64]], axis=0)
    h = jax.lax.dot_general(x64, fc1_ref[0], _CT,
                            preferred_element_type=jnp.float32)
    h = h + fc1b
    outs = []
    for b in range(_B):
        r0 = h[b * 64:b * 64 + 32]
        r1 = h[b * 64 + 32:b * 64 + 64]
        conv = (_shift_down(r0) * taps[0:1] + r0 * taps[1:2]
                + _shift_up(r0) * taps[2:3]
                + _shift_down(r1) * taps[3:4] + r1 * taps[4:5]
                + _shift_up(r1) * taps[5:6]) + dwb
        outs.append(conv[0:_E])
    g = jnp.concatenate(outs, axis=0)
    g = 0.5 * g * (1.0 + jax.lax.erf(g * 0.7071067811865476))
    y = jax.lax.dot_general(g, fc2_ref[0], _CT,
                            preferred_element_type=jnp.float32)
    y = (y + fc2b) * gs_col
    out_ref[...] = jnp.zeros((_NT, _OUT), jnp.float32)
    out_ref[0:_E, :] = y[0:_E]
    out_ref[_N:_N + _E, :] = y[_E:2 * _E]


def kernel(x, H, W, wg_w, wg_b, fc1_w, fc1_b, dw_w, dw_b, fc2_w, fc2_b):
    xf = x.reshape(_NT, _DIM)
    zrow = jnp.zeros((1, _HID - _DIM), jnp.float32)
    pp = jnp.concatenate([
        jnp.concatenate([wg_w, jnp.zeros((_E, _HID - _DIM), jnp.float32)], axis=1),
        jnp.concatenate([wg_b.reshape(1, _E),
                         jnp.zeros((1, _HID - _E), jnp.float32)], axis=1),
        fc1_b[0:1],
        dw_b[0:1],
        jnp.concatenate([fc2_b[0:1], zrow], axis=1),
    ], axis=0)
    dwt = dw_w[0, :, 0].reshape(_HID, 9)
    out = pl.pallas_call(
        _moe_kernel,
        grid=(1,),
        in_specs=[
            pl.BlockSpec((_NT, _DIM), lambda i: (0, 0)),
            pl.BlockSpec((12, _HID), lambda i: (0, 0)),
            pl.BlockSpec((_HID, 9), lambda i: (0, 0)),
            pl.BlockSpec((1, _HID, _DIM), lambda i: (0, 0, 0)),
            pl.BlockSpec((1, _OUT, _HID), lambda i: (0, 0, 0)),
        ],
        out_specs=pl.BlockSpec((_NT, _OUT), lambda i: (0, 0)),
        out_shape=jax.ShapeDtypeStruct((_NT, _OUT), jnp.float32),
    )(xf, pp, dwt, fc1_w, fc2_w)
    return (out.reshape(_B, _N, _OUT), None)


# speedup vs baseline: 1.2977x; 1.2977x over previous
import jax
import jax.numpy as jnp
from jax.experimental import pallas as pl
from jax.experimental.pallas import tpu as pltpu

_E = 8
_DIM = 96
_HID = 384
_OUT = 96
_B = 2
_N = 1024
_NT = _B * _N

_CT = (((1,), (1,)), ((), ()))


def _row_to_col(row):
    i = jax.lax.broadcasted_iota(jnp.int32, (_E, _E), 0)
    j = jax.lax.broadcasted_iota(jnp.int32, (_E, _E), 1)
    b = jnp.broadcast_to(row, (_E, _E))
    return jnp.sum(jnp.where(i == j, b, 0.0), axis=1, keepdims=True)


def _shift_down(a):
    return jnp.concatenate([jnp.zeros((1, _HID), jnp.float32), a[:-1]], axis=0)


def _shift_up(a):
    return jnp.concatenate([a[1:], jnp.zeros((1, _HID), jnp.float32)], axis=0)


def _moe_kernel(x_hbm, wg_hbm, wgb_hbm, fc1_hbm, fc1b_hbm, dw_hbm, dwb_hbm,
                fc2_hbm, fc2b_hbm, out_hbm,
                zbuf, xs, wgs, wgbs, f1s, f1bs, dws, dwbs, f2s, f2bs, ys, sem):
    zbuf[...] = jnp.zeros((_NT, _OUT), jnp.float32)
    zcp = pltpu.make_async_copy(zbuf, out_hbm, sem.at[0])
    zcp.start()
    cps = [
        pltpu.make_async_copy(x_hbm, xs, sem.at[1]),
        pltpu.make_async_copy(wg_hbm, wgs, sem.at[2]),
        pltpu.make_async_copy(wgb_hbm, wgbs, sem.at[3]),
        pltpu.make_async_copy(fc1_hbm.at[0], f1s, sem.at[4]),
        pltpu.make_async_copy(fc1b_hbm.at[0:1], f1bs, sem.at[5]),
        pltpu.make_async_copy(dw_hbm.at[0], dws, sem.at[6]),
        pltpu.make_async_copy(dwb_hbm.at[0:1], dwbs, sem.at[7]),
        pltpu.make_async_copy(fc2_hbm.at[0], f2s, sem.at[8]),
        pltpu.make_async_copy(fc2b_hbm.at[0:1], f2bs, sem.at[9]),
    ]
    for cp in cps:
        cp.start()
    for cp in cps:
        cp.wait()
    x = xs[...]
    lt = jax.lax.dot_general(wgs[...], x, _CT,
                             preferred_element_type=jnp.float32)
    lt = lt + _row_to_col(wgbs[...])
    mx = jnp.max(lt, axis=0, keepdims=True)
    iota_s = jax.lax.broadcasted_iota(jnp.int32, (_E, _NT), 0)
    idx = jnp.min(jnp.where(lt == mx, iota_s, _E), axis=0, keepdims=True)
    onehot = jnp.where(iota_s == idx, 1.0, 0.0)
    pres0 = jnp.max(onehot[:, :_N], axis=1, keepdims=True)
    pres1 = jnp.max(onehot[:, _N:], axis=1, keepdims=True)
    e0 = jnp.exp(lt[:, 0:_E] - mx[:, 0:_E])
    e1 = jnp.exp(lt[:, _N:_N + _E] - mx[:, _N:_N + _E])
    p0r0 = e0[0:1] / jnp.sum(e0, axis=0, keepdims=True)
    p0r1 = e1[0:1] / jnp.sum(e1, axis=0, keepdims=True)
    masked0 = _row_to_col(p0r0) * pres0
    masked1 = _row_to_col(p0r1) * pres1
    denom = masked0 + masked1 + 1e-6
    gs_col = jnp.concatenate([masked0 / denom, masked1 / denom],
                             axis=0) * float(_B)
    x64 = jnp.concatenate([x[0:64], x[_N:_N + 64]], axis=0)
    h = jax.lax.dot_general(x64, f1s[...], _CT,
                            preferred_element_type=jnp.float32)
    h = h + f1bs[...]
    taps = jnp.transpose(dws[...])[3:9]
    outs = []
    for b in range(_B):
        r0 = h[b * 64:b * 64 + 32]
        r1 = h[b * 64 + 32:b * 64 + 64]
        conv = (_shift_down(r0) * taps[0:1] + r0 * taps[1:2]
                + _shift_up(r0) * taps[2:3]
                + _shift_down(r1) * taps[3:4] + r1 * taps[4:5]
                + _shift_up(r1) * taps[5:6]) + dwbs[...]
        outs.append(conv[0:_E])
    g = jnp.concatenate(outs, axis=0)
    g = 0.5 * g * (1.0 + jax.lax.erf(g * 0.7071067811865476))
    y = jax.lax.dot_general(g, f2s[...], _CT,
                            preferred_element_type=jnp.float32)
    ys[...] = (y + f2bs[...]) * gs_col
    zcp.wait()
    cy0 = pltpu.make_async_copy(ys.at[pl.ds(0, _E), :],
                                out_hbm.at[pl.ds(0, _E), :], sem.at[10])
    cy1 = pltpu.make_async_copy(ys.at[pl.ds(_E, _E), :],
                                out_hbm.at[pl.ds(_N, _E), :], sem.at[11])
    cy0.start()
    cy1.start()
    cy0.wait()
    cy1.wait()


def kernel(x, H, W, wg_w, wg_b, fc1_w, fc1_b, dw_w, dw_b, fc2_w, fc2_b):
    xf = x.reshape(_NT, _DIM)
    wgb = wg_b.reshape(1, _E)
    dwf = dw_w.reshape(_E, _HID, 9)
    hbm = pl.BlockSpec(memory_space=pltpu.MemorySpace.HBM)
    out = pl.pallas_call(
        _moe_kernel,
        in_specs=[hbm] * 9,
        out_specs=pl.BlockSpec(memory_space=pltpu.MemorySpace.HBM),
        out_shape=jax.ShapeDtypeStruct((_NT, _OUT), jnp.float32),
        scratch_shapes=[
            pltpu.VMEM((_NT, _OUT), jnp.float32),
            pltpu.VMEM((_NT, _DIM), jnp.float32),
            pltpu.VMEM((_E, _DIM), jnp.float32),
            pltpu.VMEM((1, _E), jnp.float32),
            pltpu.VMEM((_HID, _DIM), jnp.float32),
            pltpu.VMEM((1, _HID), jnp.float32),
            pltpu.VMEM((_HID, 9), jnp.float32),
            pltpu.VMEM((1, _HID), jnp.float32),
            pltpu.VMEM((_OUT, _HID), jnp.float32),
            pltpu.VMEM((1, _OUT), jnp.float32),
            pltpu.VMEM((2 * _E, _OUT), jnp.float32),
            pltpu.SemaphoreType.DMA((12,)),
        ],
    )(xf, wg_w, wgb, fc1_w, fc1_b, dwf, dw_b, fc2_w, fc2_b)
    return (out.reshape(_B, _N, _OUT), None)
